# baseline (speedup 1.0000x reference)
import functools

import jax
import jax.numpy as jnp
from jax import lax
from jax.experimental import pallas as pl
from jax.experimental.pallas import tpu as pltpu
from jax.experimental.pallas import tpu_sc as plsc

_SMOOTH = 0.1
_RB = 32
_NBUF = 4
_GK = 16
_SC_FRAC = 2


def _split(nrows):
    if nrows % 512 == 0 and nrows >= 1024:
        return nrows // 2
    return 0


def _tc_body(x_hbm, out_ref, buf, sems, *, c, rb, nrows, nblocks):
    g = pl.program_id(0)

    def _issue(blk, slot):
        pltpu.make_async_copy(
            x_hbm.at[pl.ds(blk * rb, rb), :], buf.at[slot], sems.at[slot]
        ).start()

    @pl.when(g == 0)
    def _warmup():
        out_ref[0, 0] = 0.0
        for b in range(min(_NBUF, nblocks)):
            _issue(b, b)

    a = _SMOOTH / (c - 1)
    bw = (1.0 - _SMOOTH) - a
    nb = min(_NBUF, nblocks)
    for b in range(nb):
        blk = g * nb + b
        pltpu.make_async_copy(
            x_hbm.at[pl.ds(blk * rb, rb), :], buf.at[b], sems.at[b]
        ).wait()

        x = buf[b]
        s_tot = jnp.sum(x, axis=1, keepdims=True)
        m = jnp.max(x, axis=1, keepdims=True)
        e = jnp.exp(x - m)
        lse = m + jnp.log(jnp.sum(e, axis=1, keepdims=True))
        row_loss = a * (c * lse - s_tot) + bw * lse
        out_ref[0, 0] += jnp.sum(row_loss) / nrows

        @pl.when(blk + _NBUF < nblocks)
        def _refill():
            _issue(blk + _NBUF, b)


def _tc_loss(pred, tc_rows):
    nrows, c = pred.shape
    rb = _RB if tc_rows % (_RB * _NBUF) == 0 else tc_rows
    nblocks = max(tc_rows // rb, 1)
    nsteps = max(nblocks // _NBUF, 1)

    out = pl.pallas_call(
        functools.partial(_tc_body, c=c, rb=rb, nrows=nrows,
                          nblocks=nblocks),
        grid=(nsteps,),
        in_specs=[pl.BlockSpec(memory_space=pltpu.MemorySpace.HBM)],
        out_specs=pl.BlockSpec(memory_space=pltpu.SMEM),
        out_shape=jax.ShapeDtypeStruct((1, 1), jnp.float32),
        scratch_shapes=[
            pltpu.VMEM((min(_NBUF, nblocks), rb, c), jnp.float32),
            pltpu.SemaphoreType.DMA((min(_NBUF, nblocks),)),
        ],
        compiler_params=pltpu.CompilerParams(
            dimension_semantics=("arbitrary",),
        ),
    )(pred)
    return out.reshape(())


def _sc_partials(pred, r0, nsc):
    _, c = pred.shape
    nw = 32
    per_w = nsc // nw
    nchunk = c // 160
    rem = (c - nchunk * 160) // 16

    mesh = plsc.VectorSubcoreMesh(core_axis_name="c", subcore_axis_name="s")

    @functools.partial(
        pl.kernel,
        out_type=[jax.ShapeDtypeStruct((nsc * 16,), jnp.float32)] * 3,
        mesh=mesh,
        scratch_types=[
            pltpu.VMEM((c,), jnp.float32),
            pltpu.VMEM((16,), jnp.float32),
            pltpu.VMEM((16,), jnp.float32),
            pltpu.VMEM((16,), jnp.float32),
        ],
    )
    def k(x_hbm, m_out, e_out, s_out, rowbuf, mvec, svec, evec):
        nc = 2
        wid = lax.axis_index("s") * nc + lax.axis_index("c")
        base = r0 + wid * per_w

        def row_body(rr, carry):
            pltpu.sync_copy(x_hbm.at[base + rr], rowbuf)
            mvec[...] = jnp.full((16,), -jnp.inf, jnp.float32)
            svec[...] = jnp.zeros((16,), jnp.float32)
            evec[...] = jnp.zeros((16,), jnp.float32)

            def red_body(i, cy):
                mx = mvec[...]
                sm = svec[...]
                for u in range(10):
                    v = rowbuf[pl.ds((i * 10 + u) * 16, 16)]
                    mx = jnp.maximum(mx, v)
                    sm = sm + v
                mvec[...] = mx
                svec[...] = sm
                return cy

            lax.fori_loop(0, nchunk, red_body, 0)
            mx = mvec[...]
            sm = svec[...]
            for u in range(rem):
                v = rowbuf[pl.ds((nchunk * 10 + u) * 16, 16)]
                mx = jnp.maximum(mx, v)
                sm = sm + v
            mvec[...] = mx
            svec[...] = sm

            def exp_body(i, cy):
                es = evec[...]
                mxv = mvec[...]
                for u in range(10):
                    v = rowbuf[pl.ds((i * 10 + u) * 16, 16)]
                    es = es + jnp.exp(v - mxv)
                evec[...] = es
                return cy

            lax.fori_loop(0, nchunk, exp_body, 0)
            es = evec[...]
            mxv = mvec[...]
            for u in range(rem):
                v = rowbuf[pl.ds((nchunk * 10 + u) * 16, 16)]
                es = es + jnp.exp(v - mxv)
            evec[...] = es

            out_off = (wid * per_w + rr) * 16
            pltpu.sync_copy(mvec, m_out.at[pl.ds(out_off, 16)])
            pltpu.sync_copy(evec, e_out.at[pl.ds(out_off, 16)])
            pltpu.sync_copy(svec, s_out.at[pl.ds(out_off, 16)])
            return carry

        lax.fori_loop(0, per_w, row_body, 0)

    return k(pred)


def _gather_body(t_smem, *refs):
    xs, out_ref = refs[:-1], refs[-1]
    g = pl.program_id(0)

    @pl.when(g == 0)
    def _init():
        out_ref[0, 0] = 0.0

    lane = lax.broadcasted_iota(jnp.int32, (1, 128), 1)
    acc = jnp.zeros((1, 128), jnp.float32)
    for k in range(_GK):
        t_lane = t_smem[g * _GK + k] % 128
        row = xs[k][k % 8:k % 8 + 1, :]
        acc = acc + jnp.where(lane == t_lane, row, 0.0)
    out_ref[0, 0] += jnp.sum(acc)


def _target_sum(pred, target):
    nrows, _ = pred.shape
    grid = nrows // _GK

    def _mk_index_map(k):
        return lambda g, t: (g * (_GK // 8) + k // 8, t[g * _GK + k] // 128)

    out = pl.pallas_call(
        _gather_body,
        grid_spec=pltpu.PrefetchScalarGridSpec(
            num_scalar_prefetch=1,
            grid=(grid,),
            in_specs=[pl.BlockSpec((8, 128), _mk_index_map(k))
                      for k in range(_GK)],
            out_specs=pl.BlockSpec(memory_space=pltpu.SMEM),
        ),
        out_shape=jax.ShapeDtypeStruct((1, 1), jnp.float32),
        compiler_params=pltpu.CompilerParams(
            dimension_semantics=("arbitrary",),
        ),
    )(target.astype(jnp.int32), *([pred] * _GK))
    return out.reshape(())


def _combine_body(tc_ref, p_ref, m_ref, e_ref, s_ref, out_ref, *, c, nrows):
    a = _SMOOTH / (c - 1)
    bw = (1.0 - _SMOOTH) - a
    m = m_ref[...]
    mrow = jnp.max(m, axis=1, keepdims=True)
    ses = jnp.sum(e_ref[...] * jnp.exp(m - mrow), axis=1, keepdims=True)
    s_tot = jnp.sum(s_ref[...], axis=1, keepdims=True)
    lse = mrow + jnp.log(ses)
    row_loss = a * (c * lse - s_tot) + bw * lse
    out_ref[0, 0] = (tc_ref[0, 0] + jnp.sum(row_loss) / nrows
                     - bw * p_ref[0, 0] / nrows)


def _combine(tc_part, p_sum, m_sc, e_sc, s_sc, c, nrows):
    nsc = m_sc.shape[0] // 16
    out = pl.pallas_call(
        functools.partial(_combine_body, c=c, nrows=nrows),
        in_specs=[
            pl.BlockSpec(memory_space=pltpu.SMEM),
            pl.BlockSpec(memory_space=pltpu.SMEM),
            pl.BlockSpec((nsc, 16), lambda: (0, 0)),
            pl.BlockSpec((nsc, 16), lambda: (0, 0)),
            pl.BlockSpec((nsc, 16), lambda: (0, 0)),
        ],
        out_specs=pl.BlockSpec(memory_space=pltpu.SMEM),
        out_shape=jax.ShapeDtypeStruct((1, 1), jnp.float32),
    )(tc_part.reshape(1, 1), p_sum.reshape(1, 1),
      m_sc.reshape(nsc, 16), e_sc.reshape(nsc, 16), s_sc.reshape(nsc, 16))
    return out.reshape(())


def kernel(pred, target):
    nrows, c = pred.shape
    nsc = _split(nrows) if c % 16 == 0 else 0
    tc_rows = nrows - nsc

    tc_part = _tc_loss(pred, tc_rows)
    p_sum = _target_sum(pred, target)
    if nsc:
        m_sc, e_sc, s_sc = _sc_partials(pred, tc_rows, nsc)
        return _combine(tc_part, p_sum, m_sc, e_sc, s_sc, c, nrows)
    a = _SMOOTH / (c - 1)
    bw = (1.0 - _SMOOTH) - a
    return (tc_part - bw * p_sum / nrows).reshape(())


# speedup vs baseline: 1.6025x; 1.6025x over previous
import functools

import jax
import jax.numpy as jnp
from jax import lax
from jax.experimental import pallas as pl
from jax.experimental.pallas import tpu as pltpu

_SMOOTH = 0.1
_RB = 32
_NBUF = 4


def _loss_body(t_ref, x_hbm, out_ref, buf, sems, *, c, rb, nrows, nblocks):
    g = pl.program_id(0)

    def _issue(blk, slot):
        pltpu.make_async_copy(
            x_hbm.at[pl.ds(blk * rb, rb), :], buf.at[slot], sems.at[slot]
        ).start()

    @pl.when(g == 0)
    def _warmup():
        out_ref[0, 0] = 0.0
        for b in range(min(_NBUF, nblocks)):
            _issue(b, b)

    a = _SMOOTH / (c - 1)
    bw = (1.0 - _SMOOTH) - a
    col = lax.broadcasted_iota(jnp.int32, (rb, c), 1)

    for b in range(min(_NBUF, nblocks)):
        blk = g * min(_NBUF, nblocks) + b
        pltpu.make_async_copy(
            x_hbm.at[pl.ds(blk * rb, rb), :], buf.at[b], sems.at[b]
        ).wait()

        x = buf[b]
        t_col = t_ref[0, b * rb:(b + 1) * rb, :]
        p = jnp.sum(jnp.where(col == t_col, x, 0.0), axis=1, keepdims=True)
        s_tot = jnp.sum(x, axis=1, keepdims=True)
        m = jnp.max(x, axis=1, keepdims=True)
        e = jnp.exp(x - m)
        lse = m + jnp.log(jnp.sum(e, axis=1, keepdims=True))

        row_loss = a * (c * lse - s_tot) + bw * (lse - p)
        out_ref[0, 0] += jnp.sum(row_loss) / nrows

        @pl.when(blk + _NBUF < nblocks)
        def _refill():
            _issue(blk + _NBUF, b)


def kernel(pred, target):
    nrows, c = pred.shape
    rpg = _RB * _NBUF
    rb = _RB if nrows % rpg == 0 else nrows
    nblocks = nrows // rb
    nsteps = nblocks // _NBUF if nrows % rpg == 0 else 1
    if nrows % rpg != 0:
        nblocks, nsteps = 1, 1

    t3 = target.astype(jnp.int32).reshape(nsteps, nrows // nsteps, 1)

    out = pl.pallas_call(
        functools.partial(_loss_body, c=c, rb=rb, nrows=nrows,
                          nblocks=nblocks),
        grid=(nsteps,),
        in_specs=[
            pl.BlockSpec((1, nrows // nsteps, 1), lambda g: (g, 0, 0)),
            pl.BlockSpec(memory_space=pltpu.MemorySpace.HBM),
        ],
        out_specs=pl.BlockSpec(memory_space=pltpu.SMEM),
        out_shape=jax.ShapeDtypeStruct((1, 1), jnp.float32),
        scratch_shapes=[
            pltpu.VMEM((_NBUF, rb, c), jnp.float32),
            pltpu.SemaphoreType.DMA((_NBUF,)),
        ],
        compiler_params=pltpu.CompilerParams(
            dimension_semantics=("arbitrary",),
        ),
    )(t3, pred)
    return out.reshape(())
